# baseline (speedup 1.0000x reference)
import functools

import jax
import jax.numpy as jnp
from jax import lax
from jax.experimental import pallas as pl
from jax.experimental.pallas import tpu as pltpu
from jax.experimental.pallas import tpu_sc as plsc

@functools.lru_cache(maxsize=None)
def _make_gather(n_idx, d):
    info = plsc.get_sparse_core_info()
    nw = info.num_cores * info.num_subcores
    assert n_idx % nw == 0
    b_per_w = n_idx // nw
    chunk = 256
    nbuf = 3
    assert b_per_w % chunk == 0
    n_chunks = b_per_w // chunk

    mesh = plsc.VectorSubcoreMesh(core_axis_name="c", subcore_axis_name="s")

    @functools.partial(
        pl.kernel,
        out_type=jax.ShapeDtypeStruct((n_idx, d), jnp.float32),
        mesh=mesh,
        scratch_types=[
            pltpu.VMEM((b_per_w,), jnp.int32),
        ]
        + [pltpu.VMEM((chunk, d), jnp.float32) for _ in range(nbuf)]
        + [pltpu.SemaphoreType.DMA for _ in range(2 * nbuf)],
    )
    def gather(table_hbm, idx_hbm, out_hbm, idx_v, *scratch):
        bufs = scratch[:nbuf]
        gsem = scratch[nbuf : 2 * nbuf]
        osem = scratch[2 * nbuf :]
        wid = lax.axis_index("s") * info.num_cores + lax.axis_index("c")
        base = wid * b_per_w
        pltpu.sync_copy(idx_hbm.at[pl.ds(base, b_per_w)], idx_v)

        def start_gather(c, b):
            return pltpu.async_copy(
                table_hbm.at[idx_v.at[pl.ds(c * chunk, chunk)]], bufs[b], gsem[b]
            )

        def start_out(c, b):
            return pltpu.async_copy(
                bufs[b], out_hbm.at[pl.ds(base + c * chunk, chunk)], osem[b]
            )

        h_g = [None] * nbuf
        h_o = [None] * nbuf
        for c in range(min(nbuf - 1, n_chunks)):
            h_g[c] = start_gather(c, c)
        for c in range(n_chunks):
            b = c % nbuf
            pf = c + nbuf - 1
            if pf < n_chunks:
                nb = pf % nbuf
                if h_o[nb] is not None:
                    h_o[nb].wait()
                h_g[nb] = start_gather(pf, nb)
            h_g[b].wait()
            h_o[b] = start_out(c, b)
        for b in range(nbuf):
            if h_o[b] is not None:
                h_o[b].wait()

    return gather


def kernel(x, W_E):
    b, p = x.shape
    d = W_E.shape[0]
    idx = x.reshape(-1).astype(jnp.int32)
    table = W_E.T
    out = _make_gather(idx.shape[0], d)(table, idx)
    return out.reshape(b, p, d)


# speedup vs baseline: 1.0047x; 1.0047x over previous
import functools

import jax
import jax.numpy as jnp
from jax import lax
from jax.experimental import pallas as pl
from jax.experimental.pallas import tpu as pltpu
from jax.experimental.pallas import tpu_sc as plsc

@functools.lru_cache(maxsize=None)
def _make_gather(n_idx, d):
    info = plsc.get_sparse_core_info()
    nw = info.num_cores * info.num_subcores
    assert n_idx % nw == 0
    b_per_w = n_idx // nw
    chunk = 256
    nbuf = 3
    assert b_per_w % chunk == 0
    n_chunks = b_per_w // chunk

    mesh = plsc.VectorSubcoreMesh(core_axis_name="c", subcore_axis_name="s")

    @functools.partial(
        pl.kernel,
        out_type=jax.ShapeDtypeStruct((n_idx, d), jnp.float32),
        mesh=mesh,
        scratch_types=[
            pltpu.VMEM((b_per_w,), jnp.int32),
            pltpu.SemaphoreType.DMA,
        ]
        + [pltpu.VMEM((chunk, d), jnp.float32) for _ in range(nbuf)]
        + [pltpu.SemaphoreType.DMA for _ in range(2 * nbuf)],
    )
    def gather(table_hbm, idx_hbm, out_hbm, idx_v, isem, *scratch):
        bufs = scratch[:nbuf]
        gsem = scratch[nbuf : 2 * nbuf]
        osem = scratch[2 * nbuf :]
        wid = lax.axis_index("s") * info.num_cores + lax.axis_index("c")
        base = wid * b_per_w
        pltpu.sync_copy(idx_hbm.at[pl.ds(base, chunk)], idx_v.at[pl.ds(0, chunk)])
        h_idx = pltpu.async_copy(
            idx_hbm.at[pl.ds(base + chunk, b_per_w - chunk)],
            idx_v.at[pl.ds(chunk, b_per_w - chunk)],
            isem,
        )

        def start_gather(c, b):
            return pltpu.async_copy(
                table_hbm.at[idx_v.at[pl.ds(c * chunk, chunk)]], bufs[b], gsem[b]
            )

        def start_out(c, b):
            return pltpu.async_copy(
                bufs[b], out_hbm.at[pl.ds(base + c * chunk, chunk)], osem[b]
            )

        h_g = [None] * nbuf
        h_o = [None] * nbuf
        h_g[0] = start_gather(0, 0)
        h_idx.wait()
        for c in range(1, min(nbuf - 1, n_chunks)):
            h_g[c] = start_gather(c, c)
        for c in range(n_chunks):
            b = c % nbuf
            pf = c + nbuf - 1
            if pf < n_chunks:
                nb = pf % nbuf
                if h_o[nb] is not None:
                    h_o[nb].wait()
                h_g[nb] = start_gather(pf, nb)
            h_g[b].wait()
            h_o[b] = start_out(c, b)
        for b in range(nbuf):
            if h_o[b] is not None:
                h_o[b].wait()

    return gather


def kernel(x, W_E):
    b, p = x.shape
    d = W_E.shape[0]
    idx = x.reshape(-1).astype(jnp.int32)
    table = W_E.T
    out = _make_gather(idx.shape[0], d)(table, idx)
    return out.reshape(b, p, d)
